# baseline (speedup 1.0000x reference)
import functools

import jax
import jax.numpy as jnp
import numpy as np
from jax.experimental import pallas as pl
from jax.experimental.pallas import tpu as pltpu

_C = 32
_MIX = np.float32(0.01) * np.float32(1.0) / np.float32(32)
_KEEP = np.float32(1.0) - np.float32(0.01)
_TINY = np.finfo(np.float32).tiny


def _threefry_bits(flat_u32):
    u32 = jnp.uint32
    ks0 = u32(0)
    ks1 = u32(42)
    ks2 = ks0 ^ ks1 ^ u32(0x1BD11BDA)
    ks = (ks0, ks1, ks2)

    def rotl(v, d):
        return (v << u32(d)) | (v >> u32(32 - d))

    x1 = flat_u32 + ks1
    x0 = x1
    x1 = rotl(x1, 13) ^ x0
    rots = ((13, 15, 26, 6), (17, 29, 16, 24))
    sched = ((1, 2, 1), (2, 0, 2), (0, 1, 3), (1, 2, 4), (2, 0, 5))
    for g in range(5):
        for r in rots[g % 2][1 if g == 0 else 0:]:
            x0 = x0 + x1
            x1 = rotl(x1, r)
            x1 = x1 ^ x0
        a, b, inc = sched[g]
        x0 = x0 + ks[a]
        x1 = x1 + ks[b] + u32(inc)
    return x0 ^ x1


def _body(x_ref, o_ref, p_ref, q_ref, *, block_rows):
    @pl.when(pl.program_id(0) == 0)
    def _():
        pr = jax.lax.broadcasted_iota(jnp.int32, p_ref.shape, 0)
        pc = jax.lax.broadcasted_iota(jnp.int32, p_ref.shape, 1)
        p_ref[:, :] = ((pr >> 5) == pc).astype(jnp.bfloat16)
        qr = jax.lax.broadcasted_iota(jnp.int32, q_ref.shape, 0)
        qc = jax.lax.broadcasted_iota(jnp.int32, q_ref.shape, 1)
        q_ref[:, :] = (qr == (qc >> 5)).astype(jnp.float32)

    x = x_ref[:, :]

    r_iota = jax.lax.broadcasted_iota(jnp.int32, x.shape, 0)
    c_iota = jax.lax.broadcasted_iota(jnp.int32, x.shape, 1)
    base = pl.program_id(0) * (block_rows * 1024)
    flat = (base + r_iota * 1024 + c_iota).astype(jnp.uint32)
    bits = _threefry_bits(flat)
    fb = (bits >> jnp.uint32(9)) | jnp.uint32(0x3F800000)
    u = jax.lax.bitcast_convert_type(fb, jnp.float32) - jnp.float32(1.0)
    tiny = jnp.float32(_TINY)
    u = jnp.maximum(tiny, u * (jnp.float32(1.0) - tiny) + tiny)
    w = -jnp.log(u)

    e = jnp.exp(x)
    ndims = (((1,), (0,)), ((), ()))
    e_hi16 = e.astype(jnp.bfloat16)
    e_lo16 = (e - e_hi16.astype(jnp.float32)).astype(jnp.bfloat16)
    g = (jax.lax.dot_general(e_hi16, p_ref[:, :], ndims,
                             preferred_element_type=jnp.float32)
         + jax.lax.dot_general(e_lo16, p_ref[:, :], ndims,
                               preferred_element_type=jnp.float32))
    s = jax.lax.dot_general(g, q_ref[:, :], ndims,
                            precision=jax.lax.Precision.HIGHEST,
                            preferred_element_type=jnp.float32)
    d = jnp.float32(_MIX) * s + jnp.float32(_KEEP) * e

    race = (w / d).reshape(block_rows, _C, _C)
    m = jnp.min(race, axis=-1, keepdims=True)
    o_ref[:, :, :] = (race == m).astype(jnp.float32)


@jax.jit
def kernel(logits):
    rows, cols = logits.shape
    block_rows = 512
    grid = (rows // block_rows,)
    return pl.pallas_call(
        functools.partial(_body, block_rows=block_rows),
        out_shape=jax.ShapeDtypeStruct((rows, _C, _C), jnp.float32),
        grid=grid,
        in_specs=[pl.BlockSpec((block_rows, cols), lambda i: (i, 0))],
        out_specs=pl.BlockSpec((block_rows, _C, _C), lambda i: (i, 0, 0)),
        scratch_shapes=[pltpu.VMEM((cols, _C), jnp.bfloat16),
                        pltpu.VMEM((_C, cols), jnp.float32)],
    )(logits)


# speedup vs baseline: 1.0832x; 1.0832x over previous
import functools

import jax
import jax.numpy as jnp
import numpy as np
from jax.experimental import pallas as pl
from jax.experimental.pallas import tpu as pltpu

_C = 32
_MIX = np.float32(0.01) * np.float32(1.0) / np.float32(32)
_KEEP = np.float32(1.0) - np.float32(0.01)
_TINY = np.finfo(np.float32).tiny


def _threefry_bits(flat_u32):
    u32 = jnp.uint32
    ks0 = u32(0)
    ks1 = u32(42)
    ks2 = ks0 ^ ks1 ^ u32(0x1BD11BDA)
    ks = (ks0, ks1, ks2)

    def rotl(v, d):
        return (v << u32(d)) | (v >> u32(32 - d))

    x1 = flat_u32 + ks1
    x0 = x1
    x1 = rotl(x1, 13) ^ x0
    rots = ((13, 15, 26, 6), (17, 29, 16, 24))
    sched = ((1, 2, 1), (2, 0, 2), (0, 1, 3), (1, 2, 4), (2, 0, 5))
    for g in range(5):
        for r in rots[g % 2][1 if g == 0 else 0:]:
            x0 = x0 + x1
            x1 = rotl(x1, r)
            x1 = x1 ^ x0
        a, b, inc = sched[g]
        x0 = x0 + ks[a]
        x1 = x1 + ks[b] + u32(inc)
    return x0 ^ x1


def _body(x_ref, o_ref, b_ref, *, block_rows):
    @pl.when(pl.program_id(0) == 0)
    def _():
        br = jax.lax.broadcasted_iota(jnp.int32, b_ref.shape, 0)
        bc = jax.lax.broadcasted_iota(jnp.int32, b_ref.shape, 1)
        b_ref[:, :] = ((br >> 5) == (bc >> 5)).astype(jnp.bfloat16)

    x = x_ref[:, :]

    r_iota = jax.lax.broadcasted_iota(jnp.int32, x.shape, 0)
    c_iota = jax.lax.broadcasted_iota(jnp.int32, x.shape, 1)
    base = pl.program_id(0) * (block_rows * 1024)
    flat = (base + r_iota * 1024 + c_iota).astype(jnp.uint32)
    bits = _threefry_bits(flat)
    fb = (bits >> jnp.uint32(9)) | jnp.uint32(0x3F800000)
    u = jax.lax.bitcast_convert_type(fb, jnp.float32) - jnp.float32(1.0)
    tiny = jnp.float32(_TINY)
    u = jnp.maximum(tiny, u * (jnp.float32(1.0) - tiny) + tiny)
    w = -jnp.log(u)

    e = jnp.exp(x)
    ndims = (((1,), (0,)), ((), ()))
    e_hi16 = e.astype(jnp.bfloat16)
    e_lo16 = (e - e_hi16.astype(jnp.float32)).astype(jnp.bfloat16)
    s = (jax.lax.dot_general(e_hi16, b_ref[:, :], ndims,
                             preferred_element_type=jnp.float32)
         + jax.lax.dot_general(e_lo16, b_ref[:, :], ndims,
                               preferred_element_type=jnp.float32))
    d = jnp.float32(_MIX) * s + jnp.float32(_KEEP) * e

    race = (w / d).reshape(block_rows, _C, _C)
    m = jnp.min(race, axis=-1, keepdims=True)
    o_ref[:, :, :] = (race == m).astype(jnp.float32)


@jax.jit
def kernel(logits):
    rows, cols = logits.shape
    block_rows = 256
    grid = (rows // block_rows,)
    return pl.pallas_call(
        functools.partial(_body, block_rows=block_rows),
        out_shape=jax.ShapeDtypeStruct((rows, _C, _C), jnp.float32),
        grid=grid,
        in_specs=[pl.BlockSpec((block_rows, cols), lambda i: (i, 0))],
        out_specs=pl.BlockSpec((block_rows, _C, _C), lambda i: (i, 0, 0)),
        scratch_shapes=[pltpu.VMEM((cols, cols), jnp.bfloat16)],
    )(logits)
